# Initial kernel scaffold; baseline (speedup 1.0000x reference)
import jax
import jax.numpy as jnp
from jax.experimental import pallas as pl


def kernel(features, edge_index, W1, b1, W2, b2):
    raise NotImplementedError("write your pallas kernel here")


# speedup vs baseline: 13.3981x; 13.3981x over previous
import functools

import jax
import jax.numpy as jnp
from jax import lax
from jax.experimental import pallas as pl
from jax.experimental.pallas import tpu as pltpu
from jax.experimental.pallas import tpu_sc as plsc

N = 10000
F = 128
FH = F // 2
NH = 256
DO = 2
WP = 16
L = 128
NC = 2
NS = 16

def _mesh():
  return plsc.VectorSubcoreMesh(
      core_axis_name="c", subcore_axis_name="s", num_cores=NC, num_subcores=NS)

TPT = 632
NOUT = TPT * NS


def _groups_for(num_edges):
  unit = NC * NS * 8
  g = -(-num_edges // L)
  return -(-g // unit) * unit


def _sc_gather_scatter_add(y_hbm, acc_sh, sidx, didx, buf0, buf1, sem0, sem1,
                           rows_per_tile):
  def gather(r, buf, sem):
    pltpu.async_copy(y_hbm.at[sidx.at[r]], buf, sem)

  def gwait(buf, sem):
    pltpu.make_async_copy(y_hbm.at[sidx.at[0]], buf, sem).wait()

  gather(0, buf0, sem0)

  def body(i, carry):
    r0 = 2 * i
    r1 = r0 + 1
    gather(r1, buf1, sem1)
    gwait(buf0, sem0)
    pltpu.sync_copy(buf0, acc_sh.at[didx.at[r0]], add=True)

    @pl.when(r1 + 1 < rows_per_tile)
    def _():
      gather(r1 + 1, buf0, sem0)

    gwait(buf1, sem1)
    pltpu.sync_copy(buf1, acc_sh.at[didx.at[r1]], add=True)
    return carry

  lax.fori_loop(0, rows_per_tile // 2, body, 0)


def _writeout(acc_sh, out_ref, sid):
  plsc.subcore_barrier()
  pltpu.sync_copy(acc_sh.at[pl.ds(sid * TPT, TPT)],
                  out_ref.at[pl.ds(sid * TPT, TPT)])


def _make_deg(groups):
  rpt = groups // (NC * NS)

  @functools.partial(
      pl.kernel,
      out_type=jax.ShapeDtypeStruct((NC, NOUT, WP), jnp.float32),
      mesh=_mesh(),
      compiler_params=pltpu.CompilerParams(use_tc_tiling_on_sc=False),
      scratch_types=[
          pltpu.VMEM((rpt, L), jnp.int32),
          pltpu.VMEM((L, WP), jnp.float32),
          pltpu.VMEM_SHARED((NOUT, WP), jnp.float32),
          pltpu.SemaphoreType.DMA,
      ],
  )
  def deg_kernel(dst_hbm, zeros_hbm, ones_hbm, out_hbm, idx_v, ones_v, acc_sh,
                 sem):
    cid = lax.axis_index("c")
    sid = lax.axis_index("s")
    base = cid * (groups // NC) + sid * rpt
    pltpu.sync_copy(dst_hbm.at[pl.ds(base, rpt)], idx_v)
    pltpu.sync_copy(ones_hbm, ones_v)

    @pl.when(sid == 0)
    def _():
      pltpu.sync_copy(zeros_hbm, acc_sh)

    plsc.subcore_barrier()

    def body(i, carry):
      descs = [
          pltpu.async_copy(ones_v, acc_sh.at[idx_v.at[i * 8 + k]], sem,
                           add=True) for k in range(8)
      ]
      for d in descs:
        d.wait()
      return carry

    lax.fori_loop(0, rpt // 8, body, 0)
    _writeout(acc_sh, out_hbm.at[cid], sid)

  return deg_kernel


def _make_agg1(groups):
  rpt = groups // NS

  @functools.partial(
      pl.kernel,
      out_type=(jax.ShapeDtypeStruct((NOUT, FH), jnp.float32),
                jax.ShapeDtypeStruct((NOUT, FH), jnp.float32)),
      mesh=_mesh(),
      compiler_params=pltpu.CompilerParams(use_tc_tiling_on_sc=False),
      scratch_types=[
          pltpu.VMEM((rpt, L), jnp.int32),
          pltpu.VMEM((rpt, L), jnp.int32),
          pltpu.VMEM((L, FH), jnp.float32),
          pltpu.VMEM((L, FH), jnp.float32),
          pltpu.VMEM_SHARED((NOUT, FH), jnp.float32),
          pltpu.SemaphoreType.DMA,
          pltpu.SemaphoreType.DMA,
      ],
  )
  def agg1_kernel(src_hbm, dst_hbm, ylo_hbm, yhi_hbm, zeros_hbm, outlo_hbm,
                  outhi_hbm, sidx, didx, buf0, buf1, acc_sh, sem0, sem1):
    cid = lax.axis_index("c")
    sid = lax.axis_index("s")
    pltpu.sync_copy(src_hbm.at[pl.ds(sid * rpt, rpt)], sidx)
    pltpu.sync_copy(dst_hbm.at[pl.ds(sid * rpt, rpt)], didx)

    @pl.when(sid == 0)
    def _():
      pltpu.sync_copy(zeros_hbm, acc_sh)

    plsc.subcore_barrier()

    def run(y_hbm, out_ref):
      _sc_gather_scatter_add(y_hbm, acc_sh, sidx, didx, buf0, buf1, sem0,
                             sem1, rpt)
      _writeout(acc_sh, out_ref, sid)

    @pl.when(cid == 0)
    def _():
      run(ylo_hbm, outlo_hbm)

    @pl.when(cid == 1)
    def _():
      run(yhi_hbm, outhi_hbm)

  return agg1_kernel


def _make_agg2(groups):
  rpt = groups // (NC * NS)

  @functools.partial(
      pl.kernel,
      out_type=jax.ShapeDtypeStruct((NC, NOUT, WP), jnp.float32),
      mesh=_mesh(),
      compiler_params=pltpu.CompilerParams(use_tc_tiling_on_sc=False),
      scratch_types=[
          pltpu.VMEM((rpt, L), jnp.int32),
          pltpu.VMEM((rpt, L), jnp.int32),
          pltpu.VMEM((L, WP), jnp.float32),
          pltpu.VMEM((L, WP), jnp.float32),
          pltpu.VMEM_SHARED((NOUT, WP), jnp.float32),
          pltpu.SemaphoreType.DMA,
          pltpu.SemaphoreType.DMA,
      ],
  )
  def agg2_kernel(src_hbm, dst_hbm, y2_hbm, zeros_hbm, out_hbm, sidx, didx,
                  buf0, buf1, acc_sh, sem0, sem1):
    cid = lax.axis_index("c")
    sid = lax.axis_index("s")
    base = cid * (groups // NC) + sid * rpt
    pltpu.sync_copy(src_hbm.at[pl.ds(base, rpt)], sidx)
    pltpu.sync_copy(dst_hbm.at[pl.ds(base, rpt)], didx)

    @pl.when(sid == 0)
    def _():
      pltpu.sync_copy(zeros_hbm, acc_sh)

    plsc.subcore_barrier()
    _sc_gather_scatter_add(y2_hbm, acc_sh, sidx, didx, buf0, buf1, sem0, sem1,
                           rpt)
    _writeout(acc_sh, out_hbm.at[cid], sid)

  return agg2_kernel


_RB = 2000


def _prescale_body(dp_ref, x_ref, dinv_ref, ylo_ref, yhi_ref):
  deg = dp_ref[0, :, 0:1] + dp_ref[1, :, 0:1] + 1.0
  dinv = 1.0 / jnp.sqrt(jnp.clip(deg, 1.0, None))
  dinv_ref[...] = dinv
  y = x_ref[...] * dinv
  ylo_ref[...] = y[:, :FH]
  yhi_ref[...] = y[:, FH:]


def _prescale(deg_parts, x):
  return pl.pallas_call(
      _prescale_body,
      grid=(N // _RB,),
      in_specs=[
          pl.BlockSpec((NC, _RB, WP), lambda i: (0, i, 0)),
          pl.BlockSpec((_RB, F), lambda i: (i, 0)),
      ],
      out_specs=[
          pl.BlockSpec((_RB, 1), lambda i: (i, 0)),
          pl.BlockSpec((_RB, FH), lambda i: (i, 0)),
          pl.BlockSpec((_RB, FH), lambda i: (i, 0)),
      ],
      out_shape=[
          jax.ShapeDtypeStruct((N, 1), jnp.float32),
          jax.ShapeDtypeStruct((N, FH), jnp.float32),
          jax.ShapeDtypeStruct((N, FH), jnp.float32),
      ],
  )(deg_parts, x)


def _mid_body(alo_ref, ahi_ref, ylo_ref, yhi_ref, dinv_ref, w1_ref, b1_ref,
              w2_ref, y2_ref):
  dinv = dinv_ref[...]
  zlo = (alo_ref[...] + ylo_ref[...]) * dinv
  zhi = (ahi_ref[...] + yhi_ref[...]) * dinv
  z = jnp.concatenate([zlo, zhi], axis=1)
  h = jnp.dot(z, w1_ref[...], preferred_element_type=jnp.float32) + b1_ref[...]
  h = jnp.maximum(h, 0.0)
  t = jnp.dot(h, w2_ref[...], preferred_element_type=jnp.float32)
  y2 = t * dinv
  pad = jnp.zeros((y2.shape[0], WP - DO), jnp.float32)
  y2_ref[...] = jnp.concatenate([y2, pad], axis=1)


def _mid(acc_lo, acc_hi, ylo, yhi, dinv, w1, b1, w2):
  return pl.pallas_call(
      _mid_body,
      grid=(N // _RB,),
      in_specs=[
          pl.BlockSpec((_RB, FH), lambda i: (i, 0)),
          pl.BlockSpec((_RB, FH), lambda i: (i, 0)),
          pl.BlockSpec((_RB, FH), lambda i: (i, 0)),
          pl.BlockSpec((_RB, FH), lambda i: (i, 0)),
          pl.BlockSpec((_RB, 1), lambda i: (i, 0)),
          pl.BlockSpec((F, NH), lambda i: (0, 0)),
          pl.BlockSpec((1, NH), lambda i: (0, 0)),
          pl.BlockSpec((NH, DO), lambda i: (0, 0)),
      ],
      out_specs=pl.BlockSpec((_RB, WP), lambda i: (i, 0)),
      out_shape=jax.ShapeDtypeStruct((N, WP), jnp.float32),
  )(acc_lo, acc_hi, ylo, yhi, dinv, w1, b1, w2)


def _emb_body(a2_ref, y2_ref, dinv_ref, b2_ref, emb_ref, sq_ref):
  s = a2_ref[0, :, 0:DO] + a2_ref[1, :, 0:DO] + y2_ref[:, 0:DO]
  emb = s * dinv_ref[...] + b2_ref[...]
  emb_ref[...] = emb
  sq_ref[...] = jnp.sum(emb * emb, axis=1, keepdims=True)


def _emb(a2, y2, dinv, b2):
  return pl.pallas_call(
      _emb_body,
      grid=(N // _RB,),
      in_specs=[
          pl.BlockSpec((NC, _RB, WP), lambda i: (0, i, 0)),
          pl.BlockSpec((_RB, WP), lambda i: (i, 0)),
          pl.BlockSpec((_RB, 1), lambda i: (i, 0)),
          pl.BlockSpec((1, DO), lambda i: (0, 0)),
      ],
      out_specs=[
          pl.BlockSpec((_RB, DO), lambda i: (i, 0)),
          pl.BlockSpec((_RB, 1), lambda i: (i, 0)),
      ],
      out_shape=[
          jax.ShapeDtypeStruct((N, DO), jnp.float32),
          jax.ShapeDtypeStruct((N, 1), jnp.float32),
      ],
  )(a2, y2, dinv, b2)


_QR = 200


def _q_body(e_ref, aux_ref, q_ref):
  ai = e_ref[:, 0:1]
  bi = e_ref[:, 1:2]
  sqi = ai * ai + bi * bi
  aj = aux_ref[0:1, :]
  bj = aux_ref[1:2, :]
  sqj = aux_ref[2:3, :]
  d2 = sqi + sqj - 2.0 * (ai * aj + bi * bj)
  dist = jnp.sqrt(jnp.maximum(d2, 0.0))
  q_ref[...] = 1.0 / (1.0 + 0.5 * dist)


def _q(emb, aux):
  return pl.pallas_call(
      _q_body,
      grid=(N // _QR,),
      in_specs=[
          pl.BlockSpec((_QR, DO), lambda i: (i, 0)),
          pl.BlockSpec((3, N), lambda i: (0, 0)),
      ],
      out_specs=pl.BlockSpec((_QR, N), lambda i: (i, 0)),
      out_shape=jax.ShapeDtypeStruct((N, N), jnp.float32),
  )(emb, aux)


def kernel(features, edge_index, W1, b1, W2, b2):
  num_edges = edge_index.shape[1]
  groups = _groups_for(num_edges)
  ei = edge_index.astype(jnp.int32)
  npad = groups * L - num_edges
  src2 = jnp.concatenate([ei[0], jnp.zeros((npad,), jnp.int32)]).reshape(
      groups, L)
  dst2 = jnp.concatenate([ei[1], jnp.full((npad,), N, jnp.int32)]).reshape(
      groups, L)
  zeros_wp = jnp.zeros((NOUT, WP), jnp.float32)
  zeros_fh = jnp.zeros((NOUT, FH), jnp.float32)
  ones_wp = jnp.ones((L, WP), jnp.float32)

  deg_parts = _make_deg(groups)(dst2, zeros_wp, ones_wp)[:, :N]
  dinv, ylo, yhi = _prescale(deg_parts, features)
  acc_lo, acc_hi = _make_agg1(groups)(src2, dst2, ylo, yhi, zeros_fh)
  y2 = _mid(acc_lo[:N], acc_hi[:N], ylo, yhi, dinv, W1, b1.reshape(1, NH), W2)
  a2 = _make_agg2(groups)(src2, dst2, y2, zeros_wp)[:, :N]
  emb, sq = _emb(a2, y2, dinv, b2.reshape(1, DO))
  aux = jnp.concatenate([emb.T, sq.T], axis=0)
  q = _q(emb, aux)
  return (emb, q)


# speedup vs baseline: 15.7117x; 1.1727x over previous
import functools

import jax
import jax.numpy as jnp
from jax import lax
from jax.experimental import pallas as pl
from jax.experimental.pallas import tpu as pltpu
from jax.experimental.pallas import tpu_sc as plsc

N = 10000
F = 128
FH = F // 2
NH = 256
DO = 2
WP = 16
L = 128
NC = 2
NS = 16

def _mesh():
  return plsc.VectorSubcoreMesh(
      core_axis_name="c", subcore_axis_name="s", num_cores=NC, num_subcores=NS)

TPT = 632
NOUT = TPT * NS


def _groups_for(num_edges):
  unit = NC * NS * 8
  g = -(-num_edges // L)
  return -(-g // unit) * unit


NB1 = 4
NB2 = 8


def _sc_gather_scatter_add(y_hbm, acc_sh, sidx, didx, bufs, gsems, ssems,
                           rows_per_tile):
  NB = len(bufs)
  H = NB // 2

  def gather(r, k):
    pltpu.async_copy(y_hbm.at[sidx.at[r]], bufs[k], gsems[k])

  def scat(r, k):
    pltpu.async_copy(bufs[k], acc_sh.at[didx.at[r]], ssems[k], add=True)

  def gwait(k):
    pltpu.make_async_copy(y_hbm.at[sidx.at[0]], bufs[k], gsems[k]).wait()

  def swait(k):
    pltpu.make_async_copy(bufs[k], acc_sh.at[didx.at[0]], ssems[k]).wait()

  for k in range(H):
    gather(k, k)

  def body(i, carry):
    base = NB * i
    for k in range(H):
      @pl.when(i > 0)
      def _():
        swait(H + k)

      gather(base + H + k, H + k)
    for k in range(H):
      gwait(k)
      scat(base + k, k)
    for k in range(H):
      swait(k)

      @pl.when(i + 1 < rows_per_tile // NB)
      def _():
        gather(base + NB + k, k)
    for k in range(H):
      gwait(H + k)
      scat(base + H + k, H + k)
    return carry

  lax.fori_loop(0, rows_per_tile // NB, body, 0)
  for k in range(H):
    swait(H + k)


def _writeout(acc_sh, out_ref, sid):
  plsc.subcore_barrier()
  pltpu.sync_copy(acc_sh.at[pl.ds(sid * TPT, TPT)],
                  out_ref.at[pl.ds(sid * TPT, TPT)])


def _make_deg(groups):
  rpt = groups // (NC * NS)

  @functools.partial(
      pl.kernel,
      out_type=jax.ShapeDtypeStruct((NC, NOUT, WP), jnp.float32),
      mesh=_mesh(),
      compiler_params=pltpu.CompilerParams(use_tc_tiling_on_sc=False),
      scratch_types=[
          pltpu.VMEM((rpt, L), jnp.int32),
          pltpu.VMEM((L, WP), jnp.float32),
          pltpu.VMEM_SHARED((NOUT, WP), jnp.float32),
          pltpu.SemaphoreType.DMA,
      ],
  )
  def deg_kernel(dst_hbm, zeros_hbm, ones_hbm, out_hbm, idx_v, ones_v, acc_sh,
                 sem):
    cid = lax.axis_index("c")
    sid = lax.axis_index("s")
    base = cid * (groups // NC) + sid * rpt
    pltpu.sync_copy(dst_hbm.at[pl.ds(base, rpt)], idx_v)
    pltpu.sync_copy(ones_hbm, ones_v)

    @pl.when(sid == 0)
    def _():
      pltpu.sync_copy(zeros_hbm, acc_sh)

    plsc.subcore_barrier()

    def body(i, carry):
      descs = [
          pltpu.async_copy(ones_v, acc_sh.at[idx_v.at[i * 8 + k]], sem,
                           add=True) for k in range(8)
      ]
      for d in descs:
        d.wait()
      return carry

    lax.fori_loop(0, rpt // 8, body, 0)
    _writeout(acc_sh, out_hbm.at[cid], sid)

  return deg_kernel


def _make_agg1(groups):
  rpt = groups // NS

  @functools.partial(
      pl.kernel,
      out_type=(jax.ShapeDtypeStruct((NOUT, FH), jnp.float32),
                jax.ShapeDtypeStruct((NOUT, FH), jnp.float32)),
      mesh=_mesh(),
      compiler_params=pltpu.CompilerParams(use_tc_tiling_on_sc=False),
      scratch_types=(
          [pltpu.VMEM((rpt, L), jnp.int32)] * 2 +
          [pltpu.VMEM((L, FH), jnp.float32)] * NB1 +
          [pltpu.VMEM_SHARED((NOUT, FH), jnp.float32)] +
          [pltpu.SemaphoreType.DMA] * (2 * NB1)),
  )
  def agg1_kernel(src_hbm, dst_hbm, ylo_hbm, yhi_hbm, zeros_hbm, outlo_hbm,
                  outhi_hbm, sidx, didx, *sc):
    bufs = sc[:NB1]
    acc_sh = sc[NB1]
    gsems = sc[NB1 + 1:NB1 + 1 + NB1]
    ssems = sc[NB1 + 1 + NB1:]
    cid = lax.axis_index("c")
    sid = lax.axis_index("s")
    pltpu.sync_copy(src_hbm.at[pl.ds(sid * rpt, rpt)], sidx)
    pltpu.sync_copy(dst_hbm.at[pl.ds(sid * rpt, rpt)], didx)

    @pl.when(sid == 0)
    def _():
      pltpu.sync_copy(zeros_hbm, acc_sh)

    plsc.subcore_barrier()

    def run(y_hbm, out_ref):
      _sc_gather_scatter_add(y_hbm, acc_sh, sidx, didx, bufs, gsems, ssems,
                             rpt)
      _writeout(acc_sh, out_ref, sid)

    @pl.when(cid == 0)
    def _():
      run(ylo_hbm, outlo_hbm)

    @pl.when(cid == 1)
    def _():
      run(yhi_hbm, outhi_hbm)

  return agg1_kernel


def _make_agg2(groups):
  rpt = groups // (NC * NS)

  @functools.partial(
      pl.kernel,
      out_type=jax.ShapeDtypeStruct((NC, NOUT, WP), jnp.float32),
      mesh=_mesh(),
      compiler_params=pltpu.CompilerParams(use_tc_tiling_on_sc=False),
      scratch_types=(
          [pltpu.VMEM((rpt, L), jnp.int32)] * 2 +
          [pltpu.VMEM((L, WP), jnp.float32)] * NB2 +
          [pltpu.VMEM_SHARED((NOUT, WP), jnp.float32)] +
          [pltpu.SemaphoreType.DMA] * (2 * NB2)),
  )
  def agg2_kernel(src_hbm, dst_hbm, y2_hbm, zeros_hbm, out_hbm, sidx, didx,
                  *sc):
    bufs = sc[:NB2]
    acc_sh = sc[NB2]
    gsems = sc[NB2 + 1:NB2 + 1 + NB2]
    ssems = sc[NB2 + 1 + NB2:]
    cid = lax.axis_index("c")
    sid = lax.axis_index("s")
    base = cid * (groups // NC) + sid * rpt
    pltpu.sync_copy(src_hbm.at[pl.ds(base, rpt)], sidx)
    pltpu.sync_copy(dst_hbm.at[pl.ds(base, rpt)], didx)

    @pl.when(sid == 0)
    def _():
      pltpu.sync_copy(zeros_hbm, acc_sh)

    plsc.subcore_barrier()
    _sc_gather_scatter_add(y2_hbm, acc_sh, sidx, didx, bufs, gsems, ssems,
                           rpt)
    _writeout(acc_sh, out_hbm.at[cid], sid)

  return agg2_kernel


_RB = 2000


def _prescale_body(dp_ref, x_ref, dinv_ref, ylo_ref, yhi_ref):
  deg = dp_ref[0, :, 0:1] + dp_ref[1, :, 0:1] + 1.0
  dinv = 1.0 / jnp.sqrt(jnp.clip(deg, 1.0, None))
  dinv_ref[...] = dinv
  y = x_ref[...] * dinv
  ylo_ref[...] = y[:, :FH]
  yhi_ref[...] = y[:, FH:]


def _prescale(deg_parts, x):
  return pl.pallas_call(
      _prescale_body,
      grid=(N // _RB,),
      in_specs=[
          pl.BlockSpec((NC, _RB, WP), lambda i: (0, i, 0)),
          pl.BlockSpec((_RB, F), lambda i: (i, 0)),
      ],
      out_specs=[
          pl.BlockSpec((_RB, 1), lambda i: (i, 0)),
          pl.BlockSpec((_RB, FH), lambda i: (i, 0)),
          pl.BlockSpec((_RB, FH), lambda i: (i, 0)),
      ],
      out_shape=[
          jax.ShapeDtypeStruct((N, 1), jnp.float32),
          jax.ShapeDtypeStruct((N, FH), jnp.float32),
          jax.ShapeDtypeStruct((N, FH), jnp.float32),
      ],
  )(deg_parts, x)


def _mid_body(alo_ref, ahi_ref, ylo_ref, yhi_ref, dinv_ref, w1_ref, b1_ref,
              w2_ref, y2_ref):
  dinv = dinv_ref[...]
  zlo = (alo_ref[...] + ylo_ref[...]) * dinv
  zhi = (ahi_ref[...] + yhi_ref[...]) * dinv
  z = jnp.concatenate([zlo, zhi], axis=1)
  h = jnp.dot(z, w1_ref[...], preferred_element_type=jnp.float32) + b1_ref[...]
  h = jnp.maximum(h, 0.0)
  t = jnp.dot(h, w2_ref[...], preferred_element_type=jnp.float32)
  y2 = t * dinv
  pad = jnp.zeros((y2.shape[0], WP - DO), jnp.float32)
  y2_ref[...] = jnp.concatenate([y2, pad], axis=1)


def _mid(acc_lo, acc_hi, ylo, yhi, dinv, w1, b1, w2):
  return pl.pallas_call(
      _mid_body,
      grid=(N // _RB,),
      in_specs=[
          pl.BlockSpec((_RB, FH), lambda i: (i, 0)),
          pl.BlockSpec((_RB, FH), lambda i: (i, 0)),
          pl.BlockSpec((_RB, FH), lambda i: (i, 0)),
          pl.BlockSpec((_RB, FH), lambda i: (i, 0)),
          pl.BlockSpec((_RB, 1), lambda i: (i, 0)),
          pl.BlockSpec((F, NH), lambda i: (0, 0)),
          pl.BlockSpec((1, NH), lambda i: (0, 0)),
          pl.BlockSpec((NH, DO), lambda i: (0, 0)),
      ],
      out_specs=pl.BlockSpec((_RB, WP), lambda i: (i, 0)),
      out_shape=jax.ShapeDtypeStruct((N, WP), jnp.float32),
  )(acc_lo, acc_hi, ylo, yhi, dinv, w1, b1, w2)


def _emb_body(a2_ref, y2_ref, dinv_ref, b2_ref, emb_ref, row_ref, col_ref):
  s = a2_ref[0, :, 0:DO] + a2_ref[1, :, 0:DO] + y2_ref[:, 0:DO]
  emb = s * dinv_ref[...] + b2_ref[...]
  emb_ref[...] = emb
  a = emb[:, 0:1]
  b = emb[:, 1:2]
  sq = a * a + b * b
  one = jnp.ones_like(sq)
  row_ref[...] = jnp.concatenate([sq, one, -2.0 * a, -2.0 * b], axis=1)
  col_ref[...] = jnp.concatenate([one, sq, a, b], axis=1)


def _emb(a2, y2, dinv, b2):
  return pl.pallas_call(
      _emb_body,
      grid=(N // _RB,),
      in_specs=[
          pl.BlockSpec((NC, _RB, WP), lambda i: (0, i, 0)),
          pl.BlockSpec((_RB, WP), lambda i: (i, 0)),
          pl.BlockSpec((_RB, 1), lambda i: (i, 0)),
          pl.BlockSpec((1, DO), lambda i: (0, 0)),
      ],
      out_specs=[
          pl.BlockSpec((_RB, DO), lambda i: (i, 0)),
          pl.BlockSpec((_RB, 4), lambda i: (i, 0)),
          pl.BlockSpec((_RB, 4), lambda i: (i, 0)),
      ],
      out_shape=[
          jax.ShapeDtypeStruct((N, DO), jnp.float32),
          jax.ShapeDtypeStruct((N, 4), jnp.float32),
          jax.ShapeDtypeStruct((N, 4), jnp.float32),
      ],
  )(a2, y2, dinv, b2)


_QR = 200


def _q_body(row_ref, colt_ref, q_ref):
  d2 = jnp.dot(row_ref[...], colt_ref[...], preferred_element_type=jnp.float32)
  dist = jnp.sqrt(jnp.maximum(d2, 0.0))
  q_ref[...] = 1.0 / (1.0 + 0.5 * dist)


def _q(rows, colt):
  return pl.pallas_call(
      _q_body,
      grid=(N // _QR,),
      in_specs=[
          pl.BlockSpec((_QR, 4), lambda i: (i, 0)),
          pl.BlockSpec((4, N), lambda i: (0, 0)),
      ],
      out_specs=pl.BlockSpec((_QR, N), lambda i: (i, 0)),
      out_shape=jax.ShapeDtypeStruct((N, N), jnp.float32),
  )(rows, colt)


def kernel(features, edge_index, W1, b1, W2, b2):
  num_edges = edge_index.shape[1]
  groups = _groups_for(num_edges)
  ei = edge_index.astype(jnp.int32)
  npad = groups * L - num_edges
  src2 = jnp.concatenate([ei[0], jnp.zeros((npad,), jnp.int32)]).reshape(
      groups, L)
  dst2 = jnp.concatenate([ei[1], jnp.full((npad,), N, jnp.int32)]).reshape(
      groups, L)
  zeros_wp = jnp.zeros((NOUT, WP), jnp.float32)
  zeros_fh = jnp.zeros((NOUT, FH), jnp.float32)
  ones_wp = jnp.ones((L, WP), jnp.float32)

  deg_parts = _make_deg(groups)(dst2, zeros_wp, ones_wp)[:, :N]
  dinv, ylo, yhi = _prescale(deg_parts, features)
  acc_lo, acc_hi = _make_agg1(groups)(src2, dst2, ylo, yhi, zeros_fh)
  y2 = _mid(acc_lo[:N], acc_hi[:N], ylo, yhi, dinv, W1, b1.reshape(1, NH), W2)
  a2 = _make_agg2(groups)(src2, dst2, y2, zeros_wp)[:, :N]
  emb, rows, cols = _emb(a2, y2, dinv, b2.reshape(1, DO))
  q = _q(rows, cols.T)
  return (emb, q)
